# baseline (speedup 1.0000x reference)
import jax
import jax.numpy as jnp
from jax import lax
from jax.experimental import pallas as pl
from jax.experimental.pallas import tpu as pltpu
from jax.experimental.pallas import tpu_sc as plsc

N = 10000
E = 320000
C = 128
HC = C // 2
KS = 5
KT = KS * KS
AW = 80

NC = 2
NS = 16
EPT = E // NS
KB = 80
SBE = 2000
NBLK = SBE // KB
NSB = EPT // SBE
RPT = N // NS


def _mm_body(x_ref, wlo_ref, whi_ref, o_ref):
    ylo = jnp.dot(x_ref[...], wlo_ref[0], preferred_element_type=jnp.float32)
    yhi = jnp.dot(x_ref[...], whi_ref[0], preferred_element_type=jnp.float32)
    blo = lax.bitcast_convert_type(ylo, jnp.int32)
    bhi = lax.bitcast_convert_type(yhi, jnp.int32)
    rlo = lax.shift_right_logical(blo + 0x8000, 16)
    rhi = (bhi + 0x8000) & jnp.int32(-65536)
    o_ref[...] = (rhi | rlo)[None]


NQ = 24


def _klo(q):
    return jnp.where(q < 12, 2 * q, 2 * q - 23)


def _make_table(x, weight):
    xb = x.astype(jnp.bfloat16)
    wb = weight.astype(jnp.bfloat16)
    out = pl.pallas_call(
        _mm_body,
        grid=(NQ,),
        in_specs=[
            pl.BlockSpec((N, C), lambda q: (0, 0)),
            pl.BlockSpec((1, C, C), lambda q: (_klo(q), 0, 0)),
            pl.BlockSpec((1, C, C), lambda q: (_klo(q) + 1, 0, 0)),
        ],
        out_specs=pl.BlockSpec((1, N, C), lambda q: (q, 0, 0)),
        out_shape=jax.ShapeDtypeStruct((NQ, N, C), jnp.int32),
    )(xb, wb, wb)
    return out.reshape(NQ * N * 2, HC)


def _sc_body(tab, ps_t, edge_index, zrows, xrb, out_h,
             ps_v, col_v, dst_v, g_v, b_v, rows_v, msg_v, acc_v, xr_v, agg,
             gsem0, gsem1, ssem0, ssem1):
    cid = lax.axis_index("c")
    sid = lax.axis_index("s")
    base = sid * EPT

    pltpu.sync_copy(zrows, agg.at[pl.ds(sid * RPT, RPT)])
    plsc.subcore_barrier()

    lane = lax.iota(jnp.int32, 16)
    onevec = (1 - jnp.minimum(lane, 1)).astype(jnp.float32)

    def _msg_init(e, carry):
        msg_v[0, e, pl.ds(HC, 16)] = onevec
        msg_v[1, e, pl.ds(HC, 16)] = onevec
        return carry

    lax.fori_loop(0, KB, _msg_init, 0)

    def _basis(j, buf):
        off = j * KB
        for t in range(KB // 16):
            sl = pl.ds(off + t * 16, 16)
            dsl = pl.ds(t * 16, 16)
            p0 = ps_v[0, sl]
            p1 = ps_v[1, sl]
            c = col_v[sl]
            v0 = p0 * float(KS - 1)
            v1 = p1 * float(KS - 1)
            i0 = jnp.minimum(v0.astype(jnp.int32), KS - 2)
            i1 = jnp.minimum(v1.astype(jnp.int32), KS - 2)
            f0 = v0 - i0.astype(jnp.float32)
            f1 = v1 - i1.astype(jnp.float32)
            k0 = i1 * KS + i0
            k2 = k0 + KS
            par0 = k0 & 1
            j0 = lax.shift_right_logical(k0, 1)
            par2 = k2 & 1
            j2 = lax.shift_right_logical(k2, 1)
            cb = c * 2 + cid
            g_v[buf, 0, dsl] = (par0 * 12 + j0) * (2 * N) + cb
            g_v[buf, 1, dsl] = (par2 * 12 + j2) * (2 * N) + cb
            e0 = 1.0 - f0
            e1 = 1.0 - f1
            b_v[buf, 0, dsl] = e0 * e1
            b_v[buf, 1, dsl] = f0 * e1
            b_v[buf, 2, dsl] = e0 * f1
            b_v[buf, 3, dsl] = f0 * f1

    sems = (gsem0, gsem1)
    ssems = (ssem0, ssem1)

    def _fire(buf):
        for s in range(2):
            pltpu.async_copy(tab.at[g_v.at[buf, s]],
                             rows_v.at[buf, pl.ds(s * KB, KB)], sems[buf])

    def _wait(buf):
        for s in range(2):
            pltpu.make_async_copy(tab.at[g_v.at[buf, s]],
                                  rows_v.at[buf, pl.ds(s * KB, KB)],
                                  sems[buf]).wait()

    def _compute_block(j, buf):
        @pl.when(j >= 2)
        def _drain():
            pltpu.make_async_copy(msg_v.at[buf],
                                  agg.at[dst_v.at[pl.ds(0, KB)]],
                                  ssems[buf]).wait()

        @plsc.parallel_loop(0, KB, step=1, unroll=2)
        def _edge(e):
            b0 = b_v[buf, 0, pl.ds(e, 16)][0]
            b1 = b_v[buf, 1, pl.ds(e, 16)][0]
            b2 = b_v[buf, 2, pl.ds(e, 16)][0]
            b3 = b_v[buf, 3, pl.ds(e, 16)][0]
            for cc in range(HC // 16):
                sl = pl.ds(cc * 16, 16)
                w0 = rows_v[buf, e, sl]
                w1 = rows_v[buf, KB + e, sl]
                s0f = lax.bitcast_convert_type(lax.shift_left(w0, 16),
                                               jnp.float32)
                s1f = lax.bitcast_convert_type(w0 & jnp.int32(-65536),
                                               jnp.float32)
                s2f = lax.bitcast_convert_type(lax.shift_left(w1, 16),
                                               jnp.float32)
                s3f = lax.bitcast_convert_type(w1 & jnp.int32(-65536),
                                               jnp.float32)
                acc = (s0f * b0 + s1f * b1) + (s2f * b2 + s3f * b3)
                msg_v[buf, e, sl] = acc
        pltpu.async_copy(msg_v.at[buf], agg.at[dst_v.at[pl.ds(j * KB, KB)]],
                         ssems[buf], add=True)

    def _sb_body(sb, carry):
        sb_base = base + sb * SBE
        pltpu.sync_copy(ps_t.at[:, pl.ds(sb_base, SBE)], ps_v)
        pltpu.sync_copy(edge_index.at[1, pl.ds(sb_base, SBE)], col_v)
        pltpu.sync_copy(edge_index.at[0, pl.ds(sb_base, SBE)], dst_v)

        _basis(0, 0)
        _fire(0)

        def _pair_body(m, carry2):
            j = 2 * m
            _basis(j + 1, 1)
            _fire(1)
            _wait(0)
            _compute_block(j, 0)

            @pl.when(j + 2 < NBLK)
            def _prefetch0():
                _basis(j + 2, 0)
                _fire(0)

            _wait(1)
            _compute_block(j + 1, 1)
            return carry2

        lax.fori_loop(0, NBLK // 2, _pair_body, 0)
        _wait(0)
        _compute_block(NBLK - 1, 0)
        pltpu.make_async_copy(msg_v.at[0], agg.at[dst_v.at[pl.ds(0, KB)]],
                              ssems[0]).wait()
        pltpu.make_async_copy(msg_v.at[1], agg.at[dst_v.at[pl.ds(0, KB)]],
                              ssems[1]).wait()
        return carry

    lax.fori_loop(0, NSB, _sb_body, 0)

    plsc.subcore_barrier()

    CH = 125
    for q in range(RPT // CH):
        r0 = sid * RPT + q * CH
        pltpu.sync_copy(agg.at[pl.ds(r0, CH)], acc_v)
        pltpu.sync_copy(xrb.at[pl.ds(r0, CH), pl.ds(cid * HC, HC)], xr_v)

        @plsc.parallel_loop(0, CH, step=1, unroll=2)
        def _row(r):
            degs = acc_v[r, pl.ds(HC, 16)][0]
            degv = jnp.maximum(jnp.zeros((16,), jnp.float32) + degs, 1.0)
            inv = jnp.full((16,), 1.0, jnp.float32) / degv
            for cc in range(HC // 16):
                sl = pl.ds(cc * 16, 16)
                acc_v[r, sl] = acc_v[r, sl] * inv + xr_v[r, sl]

        pltpu.sync_copy(acc_v.at[:, pl.ds(0, HC)],
                        out_h.at[pl.ds(r0, CH), pl.ds(cid * HC, HC)])


def _run_sc(tab, ps_t, edge_index, zrows, xrb):
    mesh = plsc.VectorSubcoreMesh(core_axis_name="c", subcore_axis_name="s")
    f = pl.kernel(
        _sc_body,
        out_type=jax.ShapeDtypeStruct((N, C), jnp.float32),
        mesh=mesh,
        compiler_params=pltpu.CompilerParams(use_tc_tiling_on_sc=False),
        scratch_types=[
            pltpu.VMEM((2, SBE), jnp.float32),
            pltpu.VMEM((SBE,), jnp.int32),
            pltpu.VMEM((SBE,), jnp.int32),
            pltpu.VMEM((2, 2, KB), jnp.int32),
            pltpu.VMEM((2, 4, KB + 16), jnp.float32),
            pltpu.VMEM((2, 2 * KB, HC), jnp.int32),
            pltpu.VMEM((2, KB, AW), jnp.float32),
            pltpu.VMEM((125, AW), jnp.float32),
            pltpu.VMEM((125, HC), jnp.float32),
            pltpu.VMEM_SHARED((N, AW), jnp.float32),
            pltpu.SemaphoreType.DMA,
            pltpu.SemaphoreType.DMA,
            pltpu.SemaphoreType.DMA,
            pltpu.SemaphoreType.DMA,
        ],
    )
    return f(tab, ps_t, edge_index, zrows, xrb)


def _xr_body(x_ref, root_ref, bias_ref, o_ref):
    o_ref[...] = (jnp.dot(x_ref[...], root_ref[...],
                          preferred_element_type=jnp.float32)
                  + bias_ref[...])


def _root_transform(x, root, bias2d):
    return pl.pallas_call(
        _xr_body,
        grid=(1,),
        in_specs=[
            pl.BlockSpec((N, C), lambda i: (0, 0)),
            pl.BlockSpec((C, C), lambda i: (0, 0)),
            pl.BlockSpec((1, C), lambda i: (0, 0)),
        ],
        out_specs=pl.BlockSpec((N, C), lambda i: (0, 0)),
        out_shape=jax.ShapeDtypeStruct((N, C), jnp.float32),
    )(x, root, bias2d)


def kernel(x, edge_index, pseudo, weight, root, bias):
    tab = _make_table(x, weight)
    zrows = jnp.zeros((RPT, AW), jnp.float32)
    xrb = _root_transform(x, root, bias.reshape(1, C))
    return _run_sc(tab, pseudo.T, edge_index, zrows, xrb)


# speedup vs baseline: 1.0752x; 1.0752x over previous
import jax
import jax.numpy as jnp
from jax import lax
from jax.experimental import pallas as pl
from jax.experimental.pallas import tpu as pltpu
from jax.experimental.pallas import tpu_sc as plsc

N = 10000
E = 320000
C = 128
HC = C // 2
KS = 5
KT = KS * KS
AW = 80

NC = 2
NS = 16
EPT = E // NS
KB = 80
SBE = 2000
NBLK = SBE // KB
NSB = EPT // SBE
RPT = N // NS


def _mm_body(x_ref, wlo_ref, whi_ref, o_ref):
    ylo = jnp.dot(x_ref[...], wlo_ref[0], preferred_element_type=jnp.float32)
    yhi = jnp.dot(x_ref[...], whi_ref[0], preferred_element_type=jnp.float32)
    blo = lax.bitcast_convert_type(ylo, jnp.int32)
    bhi = lax.bitcast_convert_type(yhi, jnp.int32)
    rlo = lax.shift_right_logical(blo + 0x8000, 16)
    rhi = (bhi + 0x8000) & jnp.int32(-65536)
    o_ref[...] = (rhi | rlo)[None]


NQ = 24


def _klo(q):
    return jnp.where(q < 12, 2 * q, 2 * q - 23)


def _make_table(x, weight):
    out = pl.pallas_call(
        _mm_body,
        grid=(NQ,),
        in_specs=[
            pl.BlockSpec((N, C), lambda q: (0, 0)),
            pl.BlockSpec((1, C, C), lambda q: (_klo(q), 0, 0)),
            pl.BlockSpec((1, C, C), lambda q: (_klo(q) + 1, 0, 0)),
        ],
        out_specs=pl.BlockSpec((1, N, C), lambda q: (q, 0, 0)),
        out_shape=jax.ShapeDtypeStruct((NQ, N, C), jnp.int32),
    )(x, weight, weight)
    return out.reshape(NQ * N * 2, HC)


def _sc_body(tab, ps_t, edge_index, zrows, xrb, out_h,
             ps_v, col_v, dst_v, g_v, b_v, rows_v, msg_v, acc_v, xr_v, agg,
             gsem0, gsem1, ssem0, ssem1):
    cid = lax.axis_index("c")
    sid = lax.axis_index("s")
    base = sid * EPT

    pltpu.sync_copy(zrows, agg.at[pl.ds(sid * RPT, RPT)])
    plsc.subcore_barrier()

    lane = lax.iota(jnp.int32, 16)
    onevec = (1 - jnp.minimum(lane, 1)).astype(jnp.float32)

    def _msg_init(e, carry):
        msg_v[0, e, pl.ds(HC, 16)] = onevec
        msg_v[1, e, pl.ds(HC, 16)] = onevec
        return carry

    lax.fori_loop(0, KB, _msg_init, 0)

    def _basis(j, buf):
        off = j * KB
        for t in range(KB // 16):
            sl = pl.ds(off + t * 16, 16)
            dsl = pl.ds(t * 16, 16)
            p0 = ps_v[0, sl]
            p1 = ps_v[1, sl]
            c = col_v[sl]
            v0 = p0 * float(KS - 1)
            v1 = p1 * float(KS - 1)
            i0 = jnp.minimum(v0.astype(jnp.int32), KS - 2)
            i1 = jnp.minimum(v1.astype(jnp.int32), KS - 2)
            f0 = v0 - i0.astype(jnp.float32)
            f1 = v1 - i1.astype(jnp.float32)
            k0 = i1 * KS + i0
            k2 = k0 + KS
            par0 = k0 & 1
            j0 = lax.shift_right_logical(k0, 1)
            par2 = k2 & 1
            j2 = lax.shift_right_logical(k2, 1)
            cb = c * 2 + cid
            g_v[buf, 0, dsl] = (par0 * 12 + j0) * (2 * N) + cb
            g_v[buf, 1, dsl] = (par2 * 12 + j2) * (2 * N) + cb
            e0 = 1.0 - f0
            e1 = 1.0 - f1
            b_v[buf, 0, dsl] = e0 * e1
            b_v[buf, 1, dsl] = f0 * e1
            b_v[buf, 2, dsl] = e0 * f1
            b_v[buf, 3, dsl] = f0 * f1

    sems = (gsem0, gsem1)
    ssems = (ssem0, ssem1)

    def _fire(buf):
        for s in range(2):
            pltpu.async_copy(tab.at[g_v.at[buf, s]],
                             rows_v.at[buf, pl.ds(s * KB, KB)], sems[buf])

    def _wait(buf):
        for s in range(2):
            pltpu.make_async_copy(tab.at[g_v.at[buf, s]],
                                  rows_v.at[buf, pl.ds(s * KB, KB)],
                                  sems[buf]).wait()

    def _compute_block(j, buf):
        @pl.when(j >= 2)
        def _drain():
            pltpu.make_async_copy(msg_v.at[buf],
                                  agg.at[dst_v.at[pl.ds(0, KB)]],
                                  ssems[buf]).wait()

        @plsc.parallel_loop(0, KB, step=1, unroll=2)
        def _edge(e):
            b0 = b_v[buf, 0, pl.ds(e, 16)][0]
            b1 = b_v[buf, 1, pl.ds(e, 16)][0]
            b2 = b_v[buf, 2, pl.ds(e, 16)][0]
            b3 = b_v[buf, 3, pl.ds(e, 16)][0]
            for cc in range(HC // 16):
                sl = pl.ds(cc * 16, 16)
                w0 = rows_v[buf, e, sl]
                w1 = rows_v[buf, KB + e, sl]
                s0f = lax.bitcast_convert_type(lax.shift_left(w0, 16),
                                               jnp.float32)
                s1f = lax.bitcast_convert_type(w0 & jnp.int32(-65536),
                                               jnp.float32)
                s2f = lax.bitcast_convert_type(lax.shift_left(w1, 16),
                                               jnp.float32)
                s3f = lax.bitcast_convert_type(w1 & jnp.int32(-65536),
                                               jnp.float32)
                acc = (s0f * b0 + s1f * b1) + (s2f * b2 + s3f * b3)
                msg_v[buf, e, sl] = acc
        pltpu.async_copy(msg_v.at[buf], agg.at[dst_v.at[pl.ds(j * KB, KB)]],
                         ssems[buf], add=True)

    def _sb_body(sb, carry):
        sb_base = base + sb * SBE
        pltpu.sync_copy(ps_t.at[:, pl.ds(sb_base, SBE)], ps_v)
        pltpu.sync_copy(edge_index.at[1, pl.ds(sb_base, SBE)], col_v)
        pltpu.sync_copy(edge_index.at[0, pl.ds(sb_base, SBE)], dst_v)

        _basis(0, 0)
        _fire(0)

        def _pair_body(m, carry2):
            j = 2 * m
            _basis(j + 1, 1)
            _fire(1)
            _wait(0)
            _compute_block(j, 0)

            @pl.when(j + 2 < NBLK)
            def _prefetch0():
                _basis(j + 2, 0)
                _fire(0)

            _wait(1)
            _compute_block(j + 1, 1)
            return carry2

        lax.fori_loop(0, NBLK // 2, _pair_body, 0)
        _wait(0)
        _compute_block(NBLK - 1, 0)
        pltpu.make_async_copy(msg_v.at[0], agg.at[dst_v.at[pl.ds(0, KB)]],
                              ssems[0]).wait()
        pltpu.make_async_copy(msg_v.at[1], agg.at[dst_v.at[pl.ds(0, KB)]],
                              ssems[1]).wait()
        return carry

    lax.fori_loop(0, NSB, _sb_body, 0)

    plsc.subcore_barrier()

    CH = 125
    for q in range(RPT // CH):
        r0 = sid * RPT + q * CH
        pltpu.sync_copy(agg.at[pl.ds(r0, CH)], acc_v)
        pltpu.sync_copy(xrb.at[pl.ds(r0, CH), pl.ds(cid * HC, HC)], xr_v)

        @plsc.parallel_loop(0, CH, step=1, unroll=2)
        def _row(r):
            degs = acc_v[r, pl.ds(HC, 16)][0]
            degv = jnp.maximum(jnp.zeros((16,), jnp.float32) + degs, 1.0)
            inv = jnp.full((16,), 1.0, jnp.float32) / degv
            for cc in range(HC // 16):
                sl = pl.ds(cc * 16, 16)
                acc_v[r, sl] = acc_v[r, sl] * inv + xr_v[r, sl]

        pltpu.sync_copy(acc_v.at[:, pl.ds(0, HC)],
                        out_h.at[pl.ds(r0, CH), pl.ds(cid * HC, HC)])


def _run_sc(tab, ps_t, edge_index, zrows, xrb):
    mesh = plsc.VectorSubcoreMesh(core_axis_name="c", subcore_axis_name="s")
    f = pl.kernel(
        _sc_body,
        out_type=jax.ShapeDtypeStruct((N, C), jnp.float32),
        mesh=mesh,
        compiler_params=pltpu.CompilerParams(use_tc_tiling_on_sc=False),
        scratch_types=[
            pltpu.VMEM((2, SBE), jnp.float32),
            pltpu.VMEM((SBE,), jnp.int32),
            pltpu.VMEM((SBE,), jnp.int32),
            pltpu.VMEM((2, 2, KB), jnp.int32),
            pltpu.VMEM((2, 4, KB + 16), jnp.float32),
            pltpu.VMEM((2, 2 * KB, HC), jnp.int32),
            pltpu.VMEM((2, KB, AW), jnp.float32),
            pltpu.VMEM((125, AW), jnp.float32),
            pltpu.VMEM((125, HC), jnp.float32),
            pltpu.VMEM_SHARED((N, AW), jnp.float32),
            pltpu.SemaphoreType.DMA,
            pltpu.SemaphoreType.DMA,
            pltpu.SemaphoreType.DMA,
            pltpu.SemaphoreType.DMA,
        ],
    )
    return f(tab, ps_t, edge_index, zrows, xrb)


def _xr_body(x_ref, root_ref, bias_ref, o_ref):
    o_ref[...] = (jnp.dot(x_ref[...], root_ref[...],
                          preferred_element_type=jnp.float32)
                  + bias_ref[...])


def _root_transform(x, root, bias2d):
    return pl.pallas_call(
        _xr_body,
        grid=(1,),
        in_specs=[
            pl.BlockSpec((N, C), lambda i: (0, 0)),
            pl.BlockSpec((C, C), lambda i: (0, 0)),
            pl.BlockSpec((1, C), lambda i: (0, 0)),
        ],
        out_specs=pl.BlockSpec((N, C), lambda i: (0, 0)),
        out_shape=jax.ShapeDtypeStruct((N, C), jnp.float32),
    )(x, root, bias2d)


def kernel(x, edge_index, pseudo, weight, root, bias):
    tab = _make_table(x, weight)
    zrows = jnp.zeros((RPT, AW), jnp.float32)
    xrb = _root_transform(x, root, bias.reshape(1, C))
    return _run_sc(tab, pseudo.T, edge_index, zrows, xrb)


# speedup vs baseline: 1.1646x; 1.0832x over previous
import jax
import jax.numpy as jnp
from jax import lax
from jax.experimental import pallas as pl
from jax.experimental.pallas import tpu as pltpu
from jax.experimental.pallas import tpu_sc as plsc

N = 10000
E = 320000
C = 128
HC = C // 2
KS = 5
KT = KS * KS
AW = 80

NC = 2
NS = 16
EPT = E // NS
KB = 80
SBE = 2000
NBLK = SBE // KB
NSB = EPT // SBE
RPT = N // NS


def _mm_body(x_ref, wlo_ref, whi_ref, o_ref):
    ylo = jnp.dot(x_ref[...], wlo_ref[0], preferred_element_type=jnp.float32)
    yhi = jnp.dot(x_ref[...], whi_ref[0], preferred_element_type=jnp.float32)
    blo = lax.bitcast_convert_type(ylo, jnp.int32)
    bhi = lax.bitcast_convert_type(yhi, jnp.int32)
    rlo = lax.shift_right_logical(blo + 0x8000, 16)
    rhi = (bhi + 0x8000) & jnp.int32(-65536)
    o_ref[...] = (rhi | rlo)[None]


NQ = 24


def _klo(q):
    return jnp.where(q < 12, 2 * q, 2 * q - 23)


def _make_table(x, weight):
    out = pl.pallas_call(
        _mm_body,
        grid=(NQ,),
        in_specs=[
            pl.BlockSpec((N, C), lambda q: (0, 0)),
            pl.BlockSpec((1, C, C), lambda q: (_klo(q), 0, 0)),
            pl.BlockSpec((1, C, C), lambda q: (_klo(q) + 1, 0, 0)),
        ],
        out_specs=pl.BlockSpec((1, N, C), lambda q: (q, 0, 0)),
        out_shape=jax.ShapeDtypeStruct((NQ, N, C), jnp.int32),
    )(x, weight, weight)
    return out.reshape(NQ * N * 2, HC)


def _sc_body(tab, ps_t, edge_index, zrows, xrb, out_h,
             ps_v, col_v, dst_v, g_v, b_v, rows_v, msg_v, acc_v, xr_v, agg,
             gsem0, gsem1, ssem0, ssem1):
    cid = lax.axis_index("c")
    sid = lax.axis_index("s")
    base = sid * EPT

    pltpu.sync_copy(zrows, agg.at[pl.ds(sid * RPT, RPT)])
    plsc.subcore_barrier()

    lane = lax.iota(jnp.int32, 16)
    onevec = (1 - jnp.minimum(lane, 1)).astype(jnp.float32)

    def _msg_init(e, carry):
        msg_v[0, e, pl.ds(HC, 16)] = onevec
        msg_v[1, e, pl.ds(HC, 16)] = onevec
        return carry

    lax.fori_loop(0, KB, _msg_init, 0)

    def _basis(j, buf):
        off = j * KB
        for t in range(KB // 16):
            sl = pl.ds(off + t * 16, 16)
            dsl = pl.ds(t * 16, 16)
            p0 = ps_v[0, sl]
            p1 = ps_v[1, sl]
            c = col_v[sl]
            v0 = p0 * float(KS - 1)
            v1 = p1 * float(KS - 1)
            i0 = jnp.minimum(v0.astype(jnp.int32), KS - 2)
            i1 = jnp.minimum(v1.astype(jnp.int32), KS - 2)
            f0 = v0 - i0.astype(jnp.float32)
            f1 = v1 - i1.astype(jnp.float32)
            k0 = i1 * KS + i0
            k2 = k0 + KS
            par0 = k0 & 1
            j0 = lax.shift_right_logical(k0, 1)
            par2 = k2 & 1
            j2 = lax.shift_right_logical(k2, 1)
            cb = c * 2 + cid
            g_v[buf, 0, dsl] = (par0 * 12 + j0) * (2 * N) + cb
            g_v[buf, 1, dsl] = (par2 * 12 + j2) * (2 * N) + cb
            e0 = 1.0 - f0
            e1 = 1.0 - f1
            b01 = plsc.pack(e0 * e1, f0 * e1,
                            format=plsc.PackFormat.INTERLEAVED)
            b23 = plsc.pack(e0 * f1, f0 * f1,
                            format=plsc.PackFormat.INTERLEAVED)
            b_v[buf, 0, dsl] = plsc.bitcast(b01, jnp.int32)
            b_v[buf, 1, dsl] = plsc.bitcast(b23, jnp.int32)

    sems = (gsem0, gsem1)
    ssems = (ssem0, ssem1)

    def _fire(buf):
        for s in range(2):
            pltpu.async_copy(tab.at[g_v.at[buf, s]],
                             rows_v.at[buf, pl.ds(s * KB, KB)], sems[buf])

    def _wait(buf):
        for s in range(2):
            pltpu.make_async_copy(tab.at[g_v.at[buf, s]],
                                  rows_v.at[buf, pl.ds(s * KB, KB)],
                                  sems[buf]).wait()

    def _compute_block(j, buf):
        @pl.when(j >= 2)
        def _drain():
            pltpu.make_async_copy(msg_v.at[buf],
                                  agg.at[dst_v.at[pl.ds(0, KB)]],
                                  ssems[buf]).wait()

        @plsc.parallel_loop(0, KB, step=1, unroll=2)
        def _edge(e):
            zi = jnp.zeros((16,), jnp.int32)
            b01 = plsc.bitcast(zi + b_v[buf, 0, pl.ds(e, 16)][0],
                               jnp.bfloat16)
            b23 = plsc.bitcast(zi + b_v[buf, 1, pl.ds(e, 16)][0],
                               jnp.bfloat16)
            for cc in range(HC // 16):
                sl = pl.ds(cc * 16, 16)
                w0 = plsc.bitcast(rows_v[buf, e, sl], jnp.bfloat16)
                w1 = plsc.bitcast(rows_v[buf, KB + e, sl], jnp.bfloat16)
                t = w0 * b01 + w1 * b23
                lo, hi = plsc.unpack(t, format=plsc.PackFormat.INTERLEAVED)
                msg_v[buf, e, sl] = lo + hi
        pltpu.async_copy(msg_v.at[buf], agg.at[dst_v.at[pl.ds(j * KB, KB)]],
                         ssems[buf], add=True)

    def _sb_body(sb, carry):
        sb_base = base + sb * SBE
        pltpu.sync_copy(ps_t.at[:, pl.ds(sb_base, SBE)], ps_v)
        pltpu.sync_copy(edge_index.at[1, pl.ds(sb_base, SBE)], col_v)
        pltpu.sync_copy(edge_index.at[0, pl.ds(sb_base, SBE)], dst_v)

        _basis(0, 0)
        _fire(0)

        def _pair_body(m, carry2):
            j = 2 * m
            _basis(j + 1, 1)
            _fire(1)
            _wait(0)
            _compute_block(j, 0)

            @pl.when(j + 2 < NBLK)
            def _prefetch0():
                _basis(j + 2, 0)
                _fire(0)

            _wait(1)
            _compute_block(j + 1, 1)
            return carry2

        lax.fori_loop(0, NBLK // 2, _pair_body, 0)
        _wait(0)
        _compute_block(NBLK - 1, 0)
        pltpu.make_async_copy(msg_v.at[0], agg.at[dst_v.at[pl.ds(0, KB)]],
                              ssems[0]).wait()
        pltpu.make_async_copy(msg_v.at[1], agg.at[dst_v.at[pl.ds(0, KB)]],
                              ssems[1]).wait()
        return carry

    lax.fori_loop(0, NSB, _sb_body, 0)

    plsc.subcore_barrier()

    CH = 125
    for q in range(RPT // CH):
        r0 = sid * RPT + q * CH
        pltpu.sync_copy(agg.at[pl.ds(r0, CH)], acc_v)
        pltpu.sync_copy(xrb.at[pl.ds(r0, CH), pl.ds(cid * HC, HC)], xr_v)

        @plsc.parallel_loop(0, CH, step=1, unroll=2)
        def _row(r):
            degs = acc_v[r, pl.ds(HC, 16)][0]
            degv = jnp.maximum(jnp.zeros((16,), jnp.float32) + degs, 1.0)
            inv = jnp.full((16,), 1.0, jnp.float32) / degv
            for cc in range(HC // 16):
                sl = pl.ds(cc * 16, 16)
                acc_v[r, sl] = acc_v[r, sl] * inv + xr_v[r, sl]

        pltpu.sync_copy(acc_v.at[:, pl.ds(0, HC)],
                        out_h.at[pl.ds(r0, CH), pl.ds(cid * HC, HC)])


def _run_sc(tab, ps_t, edge_index, zrows, xrb):
    mesh = plsc.VectorSubcoreMesh(core_axis_name="c", subcore_axis_name="s")
    f = pl.kernel(
        _sc_body,
        out_type=jax.ShapeDtypeStruct((N, C), jnp.float32),
        mesh=mesh,
        compiler_params=pltpu.CompilerParams(use_tc_tiling_on_sc=False,
                                             needs_layout_passes=False),
        scratch_types=[
            pltpu.VMEM((2, SBE), jnp.float32),
            pltpu.VMEM((SBE,), jnp.int32),
            pltpu.VMEM((SBE,), jnp.int32),
            pltpu.VMEM((2, 2, KB), jnp.int32),
            pltpu.VMEM((2, 2, KB + 16), jnp.int32),
            pltpu.VMEM((2, 2 * KB, HC), jnp.int32),
            pltpu.VMEM((2, KB, AW), jnp.float32),
            pltpu.VMEM((125, AW), jnp.float32),
            pltpu.VMEM((125, HC), jnp.float32),
            pltpu.VMEM_SHARED((N, AW), jnp.float32),
            pltpu.SemaphoreType.DMA,
            pltpu.SemaphoreType.DMA,
            pltpu.SemaphoreType.DMA,
            pltpu.SemaphoreType.DMA,
        ],
    )
    return f(tab, ps_t, edge_index, zrows, xrb)


def _xr_body(x_ref, root_ref, bias_ref, o_ref):
    o_ref[...] = (jnp.dot(x_ref[...], root_ref[...],
                          preferred_element_type=jnp.float32)
                  + bias_ref[...])


def _root_transform(x, root, bias2d):
    return pl.pallas_call(
        _xr_body,
        grid=(1,),
        in_specs=[
            pl.BlockSpec((N, C), lambda i: (0, 0)),
            pl.BlockSpec((C, C), lambda i: (0, 0)),
            pl.BlockSpec((1, C), lambda i: (0, 0)),
        ],
        out_specs=pl.BlockSpec((N, C), lambda i: (0, 0)),
        out_shape=jax.ShapeDtypeStruct((N, C), jnp.float32),
    )(x, root, bias2d)


def kernel(x, edge_index, pseudo, weight, root, bias):
    tab = _make_table(x, weight)
    zrows = jnp.zeros((RPT, AW), jnp.float32)
    xrb = _root_transform(x, root, bias.reshape(1, C))
    return _run_sc(tab, pseudo.T, edge_index, zrows, xrb)


# speedup vs baseline: 1.2183x; 1.0461x over previous
import jax
import jax.numpy as jnp
from jax import lax
from jax.experimental import pallas as pl
from jax.experimental.pallas import tpu as pltpu
from jax.experimental.pallas import tpu_sc as plsc

N = 10000
E = 320000
C = 128
HC = C // 2
KS = 5
KT = KS * KS
AW = 80

NC = 2
NS = 16
EPT = E // NS
KB = 80
SBE = 4000
NBLK = SBE // KB
NSB = EPT // SBE
RPT = N // NS


def _mm_body(x_ref, wlo_ref, whi_ref, o_ref):
    ylo = jnp.dot(x_ref[...], wlo_ref[0], preferred_element_type=jnp.float32)
    yhi = jnp.dot(x_ref[...], whi_ref[0], preferred_element_type=jnp.float32)
    blo = lax.bitcast_convert_type(ylo, jnp.int32)
    bhi = lax.bitcast_convert_type(yhi, jnp.int32)
    rlo = lax.shift_right_logical(blo + 0x8000, 16)
    rhi = (bhi + 0x8000) & jnp.int32(-65536)
    o_ref[...] = (rhi | rlo)[None]


NQ = 24


def _klo(q):
    return jnp.where(q < 12, 2 * q, 2 * q - 23)


def _make_table(x, weight):
    out = pl.pallas_call(
        _mm_body,
        grid=(NQ,),
        in_specs=[
            pl.BlockSpec((N, C), lambda q: (0, 0)),
            pl.BlockSpec((1, C, C), lambda q: (_klo(q), 0, 0)),
            pl.BlockSpec((1, C, C), lambda q: (_klo(q) + 1, 0, 0)),
        ],
        out_specs=pl.BlockSpec((1, N, C), lambda q: (q, 0, 0)),
        out_shape=jax.ShapeDtypeStruct((NQ, N, C), jnp.int32),
    )(x, weight, weight)
    return out.reshape(NQ * N * 2, HC)


def _sc_body(tab, ps_t, edge_index, zrows, xrb, out_h,
             ps_v, col_v, dst_v, g_v, b_v, rows_v, msg_v, acc_v, xr_v, agg,
             gsem0, gsem1, ssem0, ssem1):
    cid = lax.axis_index("c")
    sid = lax.axis_index("s")
    base = sid * EPT

    pltpu.sync_copy(zrows, agg.at[pl.ds(sid * RPT, RPT)])
    plsc.subcore_barrier()

    lane = lax.iota(jnp.int32, 16)
    onevec = (1 - jnp.minimum(lane, 1)).astype(jnp.float32)

    def _msg_init(e, carry):
        msg_v[0, e, pl.ds(HC, 16)] = onevec
        msg_v[1, e, pl.ds(HC, 16)] = onevec
        return carry

    lax.fori_loop(0, KB, _msg_init, 0)

    def _basis(j, buf):
        off = j * KB
        for t in range(KB // 16):
            sl = pl.ds(off + t * 16, 16)
            dsl = pl.ds(t * 16, 16)
            p0 = ps_v[0, sl]
            p1 = ps_v[1, sl]
            c = col_v[sl]
            v0 = p0 * float(KS - 1)
            v1 = p1 * float(KS - 1)
            i0 = jnp.minimum(v0.astype(jnp.int32), KS - 2)
            i1 = jnp.minimum(v1.astype(jnp.int32), KS - 2)
            f0 = v0 - i0.astype(jnp.float32)
            f1 = v1 - i1.astype(jnp.float32)
            k0 = i1 * KS + i0
            k2 = k0 + KS
            par0 = k0 & 1
            j0 = lax.shift_right_logical(k0, 1)
            par2 = k2 & 1
            j2 = lax.shift_right_logical(k2, 1)
            cb = c * 2 + cid
            g_v[buf, 0, dsl] = (par0 * 12 + j0) * (2 * N) + cb
            g_v[buf, 1, dsl] = (par2 * 12 + j2) * (2 * N) + cb
            e0 = 1.0 - f0
            e1 = 1.0 - f1
            b01 = plsc.pack(e0 * e1, f0 * e1,
                            format=plsc.PackFormat.INTERLEAVED)
            b23 = plsc.pack(e0 * f1, f0 * f1,
                            format=plsc.PackFormat.INTERLEAVED)
            b_v[buf, 0, dsl] = plsc.bitcast(b01, jnp.int32)
            b_v[buf, 1, dsl] = plsc.bitcast(b23, jnp.int32)

    sems = (gsem0, gsem1)
    ssems = (ssem0, ssem1)

    def _fire(buf):
        for s in range(2):
            pltpu.async_copy(tab.at[g_v.at[buf, s]],
                             rows_v.at[buf, pl.ds(s * KB, KB)], sems[buf])

    def _wait(buf):
        for s in range(2):
            pltpu.make_async_copy(tab.at[g_v.at[buf, s]],
                                  rows_v.at[buf, pl.ds(s * KB, KB)],
                                  sems[buf]).wait()

    def _compute_block(j, buf):
        @pl.when(j >= 2)
        def _drain():
            pltpu.make_async_copy(msg_v.at[buf],
                                  agg.at[dst_v.at[pl.ds(0, KB)]],
                                  ssems[buf]).wait()

        @plsc.parallel_loop(0, KB, step=1, unroll=2)
        def _edge(e):
            zi = jnp.zeros((16,), jnp.int32)
            b01 = plsc.bitcast(zi + b_v[buf, 0, pl.ds(e, 16)][0],
                               jnp.bfloat16)
            b23 = plsc.bitcast(zi + b_v[buf, 1, pl.ds(e, 16)][0],
                               jnp.bfloat16)
            for cc in range(HC // 16):
                sl = pl.ds(cc * 16, 16)
                w0 = plsc.bitcast(rows_v[buf, e, sl], jnp.bfloat16)
                w1 = plsc.bitcast(rows_v[buf, KB + e, sl], jnp.bfloat16)
                t = w0 * b01 + w1 * b23
                lo, hi = plsc.unpack(t, format=plsc.PackFormat.INTERLEAVED)
                msg_v[buf, e, sl] = lo + hi
        pltpu.async_copy(msg_v.at[buf], agg.at[dst_v.at[pl.ds(j * KB, KB)]],
                         ssems[buf], add=True)

    def _sb_body(sb, carry):
        sb_base = base + sb * SBE
        pltpu.sync_copy(ps_t.at[:, pl.ds(sb_base, SBE)], ps_v)
        pltpu.sync_copy(edge_index.at[1, pl.ds(sb_base, SBE)], col_v)
        pltpu.sync_copy(edge_index.at[0, pl.ds(sb_base, SBE)], dst_v)

        _basis(0, 0)
        _fire(0)

        def _pair_body(m, carry2):
            j = 2 * m
            _basis(j + 1, 1)
            _fire(1)
            _wait(0)
            _compute_block(j, 0)

            @pl.when(j + 2 < NBLK)
            def _prefetch0():
                _basis(j + 2, 0)
                _fire(0)

            _wait(1)
            _compute_block(j + 1, 1)
            return carry2

        lax.fori_loop(0, NBLK // 2, _pair_body, 0)
        if NBLK % 2 == 1:
            _wait(0)
            _compute_block(NBLK - 1, 0)
        pltpu.make_async_copy(msg_v.at[0], agg.at[dst_v.at[pl.ds(0, KB)]],
                              ssems[0]).wait()
        pltpu.make_async_copy(msg_v.at[1], agg.at[dst_v.at[pl.ds(0, KB)]],
                              ssems[1]).wait()
        return carry

    lax.fori_loop(0, NSB, _sb_body, 0)

    plsc.subcore_barrier()

    CH = 125
    for q in range(RPT // CH):
        r0 = sid * RPT + q * CH
        pltpu.sync_copy(agg.at[pl.ds(r0, CH)], acc_v)
        pltpu.sync_copy(xrb.at[pl.ds(r0, CH), pl.ds(cid * HC, HC)], xr_v)

        @plsc.parallel_loop(0, CH, step=1, unroll=2)
        def _row(r):
            degs = acc_v[r, pl.ds(HC, 16)][0]
            degv = jnp.maximum(jnp.zeros((16,), jnp.float32) + degs, 1.0)
            inv = jnp.full((16,), 1.0, jnp.float32) / degv
            for cc in range(HC // 16):
                sl = pl.ds(cc * 16, 16)
                acc_v[r, sl] = acc_v[r, sl] * inv + xr_v[r, sl]

        pltpu.sync_copy(acc_v.at[:, pl.ds(0, HC)],
                        out_h.at[pl.ds(r0, CH), pl.ds(cid * HC, HC)])


def _run_sc(tab, ps_t, edge_index, zrows, xrb):
    mesh = plsc.VectorSubcoreMesh(core_axis_name="c", subcore_axis_name="s")
    f = pl.kernel(
        _sc_body,
        out_type=jax.ShapeDtypeStruct((N, C), jnp.float32),
        mesh=mesh,
        compiler_params=pltpu.CompilerParams(use_tc_tiling_on_sc=False,
                                             needs_layout_passes=False),
        scratch_types=[
            pltpu.VMEM((2, SBE), jnp.float32),
            pltpu.VMEM((SBE,), jnp.int32),
            pltpu.VMEM((SBE,), jnp.int32),
            pltpu.VMEM((2, 2, KB), jnp.int32),
            pltpu.VMEM((2, 2, KB + 16), jnp.int32),
            pltpu.VMEM((2, 2 * KB, HC), jnp.int32),
            pltpu.VMEM((2, KB, AW), jnp.float32),
            pltpu.VMEM((125, AW), jnp.float32),
            pltpu.VMEM((125, HC), jnp.float32),
            pltpu.VMEM_SHARED((N, AW), jnp.float32),
            pltpu.SemaphoreType.DMA,
            pltpu.SemaphoreType.DMA,
            pltpu.SemaphoreType.DMA,
            pltpu.SemaphoreType.DMA,
        ],
    )
    return f(tab, ps_t, edge_index, zrows, xrb)


def _xr_body(x_ref, root_ref, bias_ref, o_ref):
    o_ref[...] = (jnp.dot(x_ref[...], root_ref[...],
                          preferred_element_type=jnp.float32)
                  + bias_ref[...])


def _root_transform(x, root, bias2d):
    return pl.pallas_call(
        _xr_body,
        grid=(1,),
        in_specs=[
            pl.BlockSpec((N, C), lambda i: (0, 0)),
            pl.BlockSpec((C, C), lambda i: (0, 0)),
            pl.BlockSpec((1, C), lambda i: (0, 0)),
        ],
        out_specs=pl.BlockSpec((N, C), lambda i: (0, 0)),
        out_shape=jax.ShapeDtypeStruct((N, C), jnp.float32),
    )(x, root, bias2d)


def kernel(x, edge_index, pseudo, weight, root, bias):
    tab = _make_table(x, weight)
    zrows = jnp.zeros((RPT, AW), jnp.float32)
    xrb = _root_transform(x, root, bias.reshape(1, C))
    return _run_sc(tab, pseudo.T, edge_index, zrows, xrb)
